# baseline (speedup 1.0000x reference)
import jax
import jax.numpy as jnp
from jax import lax
from jax.experimental import pallas as pl
from jax.experimental.pallas import tpu as pltpu
from jax.experimental.pallas import tpu_sc as plsc

_N_CLASSES = 100000
_SMOOTHING = 0.1
_CONFIDENCE = 1.0 - _SMOOTHING
_EPS = _SMOOTHING / (_N_CLASSES - 1)

_ROWS = 1024

_C_SC = 51200
_TC_W = 2048
_TC_BLK0 = _C_SC // _TC_W
_TC_NBLK = (_N_CLASSES - _C_SC + _TC_W - 1) // _TC_W

_NC = 2
_NS = 16
_NW = _NC * _NS
_GROUPS_PER_W = _ROWS // (8 * _NW)
_CHUNK = 6400
_NCH = _C_SC // _CHUNK
_K = _GROUPS_PER_W * _NCH
_UNROLL = 25
_INNER = _CHUNK // (16 * _UNROLL)


def _tc_kernel(lsm_ref, tgt_ref, out_ref):
    j = pl.program_id(0)
    blk = lsm_ref[...]
    col = jax.lax.broadcasted_iota(jnp.int32, (_ROWS, _TC_W), 1) + (
        _C_SC + j * _TC_W
    )
    blk = jnp.where(col < _N_CLASSES, blk, 0.0)
    s = jnp.sum(blk)
    tgt = tgt_ref[...]
    g = jnp.sum(jnp.where(col == tgt, blk, 0.0))
    out_ref[...] = jnp.reshape(_EPS * s + (_CONFIDENCE - _EPS) * g, (1, 1, 1))


def _tc_call(lsm, tgt2d):
    return pl.pallas_call(
        _tc_kernel,
        grid=(_TC_NBLK,),
        in_specs=[
            pl.BlockSpec((_ROWS, _TC_W), lambda j: (0, _TC_BLK0 + j)),
            pl.BlockSpec((_ROWS, 1), lambda j: (0, 0)),
        ],
        out_specs=pl.BlockSpec((1, 1, 1), lambda j: (j, 0, 0)),
        out_shape=jax.ShapeDtypeStruct((_TC_NBLK, 1, 1), jnp.float32),
        compiler_params=pltpu.CompilerParams(
            dimension_semantics=("arbitrary",),
        ),
    )(lsm, tgt2d)


def _sum_slab(buf, acc):
    def row_body(rr, acc):
        def inner(i, acc):
            b = i * (16 * _UNROLL)
            for u in range(_UNROLL):
                acc = acc + buf[rr, pl.ds(b + u * 16, 16)]
            return acc
        return lax.fori_loop(0, _INNER, inner, acc)
    return lax.fori_loop(0, 8, row_body, acc)


def _gather_slab(buf, tvbuf, lane, gr, c0, acc_g):
    t_vec = tvbuf[pl.ds(gr * 8, 16)]
    for rr in range(8):
        rel = t_vec[rr] - c0
        inb = (rel >= 0) & (rel < _CHUNK)
        relc = jnp.clip(rel, 0, _CHUNK - 1)
        start = (relc // 16) * 16
        sub = jnp.where(inb, relc - start, 16)
        v = buf[rr, pl.ds(start, 16)]
        acc_g = acc_g + jnp.where(lane == sub, v, 0.0)
    return acc_g


def _slab_src(lsm_hbm, k, wid):
    gr = wid * _GROUPS_PER_W + k // _NCH
    c0 = (k % _NCH) * _CHUNK
    row0 = pl.multiple_of(gr * 8, 8)
    c0 = pl.multiple_of(c0, 128)
    return lsm_hbm.at[pl.ds(row0, 8), pl.ds(c0, _CHUNK)], c0, gr


def _sc_body(lsm_hbm, tgt_hbm, out_hbm, tvbuf, buf0, buf1, outbuf, sem0, sem1):
    c = lax.axis_index("c")
    s = lax.axis_index("s")
    wid = s * _NC + c
    pltpu.sync_copy(tgt_hbm, tvbuf.at[pl.ds(0, _ROWS)])
    lane = lax.iota(jnp.int32, 16)

    src0, _, _ = _slab_src(lsm_hbm, 0, wid)
    src1, _, _ = _slab_src(lsm_hbm, 1, wid)
    pltpu.make_async_copy(src0, buf0, sem0).start()
    pltpu.make_async_copy(src1, buf1, sem1).start()

    def pair_step(p, carry):
        acc_s, acc_g = carry
        k = p * 2

        src_a, c0_a, gr_a = _slab_src(lsm_hbm, k, wid)
        pltpu.make_async_copy(src_a, buf0, sem0).wait()
        acc_s = _sum_slab(buf0, acc_s)
        acc_g = _gather_slab(buf0, tvbuf, lane, gr_a, c0_a, acc_g)

        @pl.when(k + 2 < _K)
        def _():
            src_n, _, _ = _slab_src(lsm_hbm, k + 2, wid)
            pltpu.make_async_copy(src_n, buf0, sem0).start()

        src_b, c0_b, gr_b = _slab_src(lsm_hbm, k + 1, wid)
        pltpu.make_async_copy(src_b, buf1, sem1).wait()
        acc_s = _sum_slab(buf1, acc_s)
        acc_g = _gather_slab(buf1, tvbuf, lane, gr_b, c0_b, acc_g)

        @pl.when(k + 3 < _K)
        def _():
            src_n, _, _ = _slab_src(lsm_hbm, k + 3, wid)
            pltpu.make_async_copy(src_n, buf1, sem1).start()

        return acc_s, acc_g

    zero = jnp.zeros((16,), jnp.float32)
    acc_s, acc_g = lax.fori_loop(0, _K // 2, pair_step, (zero, zero))
    outbuf[...] = _EPS * acc_s + (_CONFIDENCE - _EPS) * acc_g
    pltpu.sync_copy(outbuf, out_hbm.at[pl.ds(wid * 16, 16)])


_sc_call = pl.kernel(
    _sc_body,
    out_type=jax.ShapeDtypeStruct((_NW * 16,), jnp.float32),
    mesh=plsc.VectorSubcoreMesh(
        core_axis_name="c", subcore_axis_name="s", num_cores=_NC, num_subcores=_NS
    ),
    scratch_types=[
        pltpu.VMEM((_ROWS + 16,), jnp.int32),
        pltpu.VMEM((8, _CHUNK), jnp.float32),
        pltpu.VMEM((8, _CHUNK), jnp.float32),
        pltpu.VMEM((16,), jnp.float32),
        pltpu.SemaphoreType.DMA,
        pltpu.SemaphoreType.DMA,
    ],
    compiler_params=pltpu.CompilerParams(use_tc_tiling_on_sc=True),
)


def kernel(lsm, target):
    tgt = target.astype(jnp.int32)
    sc_partials = _sc_call(lsm, tgt)
    tc_partials = _tc_call(lsm, tgt.reshape(_ROWS, 1))
    return -(jnp.sum(sc_partials) + jnp.sum(tc_partials)) / _ROWS


# speedup vs baseline: 3.3170x; 3.3170x over previous
import jax
import jax.numpy as jnp
from jax import lax
from jax.experimental import pallas as pl
from jax.experimental.pallas import tpu as pltpu
from jax.experimental.pallas import tpu_sc as plsc

_N_CLASSES = 100000
_SMOOTHING = 0.1
_CONFIDENCE = 1.0 - _SMOOTHING
_EPS = _SMOOTHING / (_N_CLASSES - 1)

_ROWS = 1024

_C_SC = 49152
_TC_BR = 2048
_TC_BLK0 = _C_SC // _TC_BR
_TC_NBLK = (_N_CLASSES - _C_SC + _TC_BR - 1) // _TC_BR

_NC = 2
_NS = 16
_NW = _NC * _NS
_CPW = _C_SC // _NW
_SLAB = 48
_NSLAB = _CPW // _SLAB
_RG = _ROWS // 16


def _tc_kernel(lsmt_ref, tgt_ref, out_ref):
    j = pl.program_id(0)
    blk = lsmt_ref[...]
    cls = jax.lax.broadcasted_iota(jnp.int32, (_TC_BR, _ROWS), 0) + (
        _C_SC + j * _TC_BR
    )
    blk0 = jnp.where(cls < _N_CLASSES, blk, 0.0)
    s = jnp.sum(blk0)
    tgt = tgt_ref[...]
    g = jnp.sum(jnp.where(cls == tgt, blk, 0.0))
    out_ref[...] = jnp.reshape(_EPS * s + (_CONFIDENCE - _EPS) * g, (1, 1, 1))


def _tc_call(lsmt, tgt2d):
    return pl.pallas_call(
        _tc_kernel,
        grid=(_TC_NBLK,),
        in_specs=[
            pl.BlockSpec((_TC_BR, _ROWS), lambda j: (_TC_BLK0 + j, 0)),
            pl.BlockSpec((1, _ROWS), lambda j: (0, 0)),
        ],
        out_specs=pl.BlockSpec((1, 1, 1), lambda j: (j, 0, 0)),
        out_shape=jax.ShapeDtypeStruct((_TC_NBLK, 1, 1), jnp.float32),
        compiler_params=pltpu.CompilerParams(
            dimension_semantics=("arbitrary",),
        ),
    )(lsmt, tgt2d)


def _process_slab(buf, tvbuf, c0, carry):
    def rg_body(rg, carry):
        acc_s, acc_g = carry
        t_slice = tvbuf[pl.ds(rg * 16, 16)]
        for col in range(_SLAB):
            v = buf[col, pl.ds(rg * 16, 16)]
            acc_s = acc_s + v
            acc_g = acc_g + jnp.where(t_slice == c0 + col, v, 0.0)
        return acc_s, acc_g
    return lax.fori_loop(0, _RG, rg_body, carry)


def _sc_body(lsmt_hbm, tgt_hbm, out_hbm, tvbuf, buf0, buf1, outbuf, sem0, sem1):
    c = lax.axis_index("c")
    s = lax.axis_index("s")
    wid = s * _NC + c
    cbase = wid * _CPW
    pltpu.sync_copy(tgt_hbm, tvbuf)

    def slab_src(k):
        c0 = pl.multiple_of(cbase + k * _SLAB, 8)
        return lsmt_hbm.at[pl.ds(c0, _SLAB), :], c0

    src0, _ = slab_src(0)
    src1, _ = slab_src(1)
    pltpu.make_async_copy(src0, buf0, sem0).start()
    pltpu.make_async_copy(src1, buf1, sem1).start()

    def pair_step(p, carry):
        k = p * 2

        src_a, c0_a = slab_src(k)
        pltpu.make_async_copy(src_a, buf0, sem0).wait()
        carry = _process_slab(buf0, tvbuf, c0_a, carry)

        @pl.when(k + 2 < _NSLAB)
        def _():
            src_n, _ = slab_src(k + 2)
            pltpu.make_async_copy(src_n, buf0, sem0).start()

        src_b, c0_b = slab_src(k + 1)
        pltpu.make_async_copy(src_b, buf1, sem1).wait()
        carry = _process_slab(buf1, tvbuf, c0_b, carry)

        @pl.when(k + 3 < _NSLAB)
        def _():
            src_n, _ = slab_src(k + 3)
            pltpu.make_async_copy(src_n, buf1, sem1).start()

        return carry

    zero = jnp.zeros((16,), jnp.float32)
    acc_s, acc_g = lax.fori_loop(0, _NSLAB // 2, pair_step, (zero, zero))
    outbuf[...] = _EPS * acc_s + (_CONFIDENCE - _EPS) * acc_g
    pltpu.sync_copy(outbuf, out_hbm.at[pl.ds(wid * 16, 16)])


_sc_call = pl.kernel(
    _sc_body,
    out_type=jax.ShapeDtypeStruct((_NW * 16,), jnp.float32),
    mesh=plsc.VectorSubcoreMesh(
        core_axis_name="c", subcore_axis_name="s", num_cores=_NC, num_subcores=_NS
    ),
    scratch_types=[
        pltpu.VMEM((_ROWS,), jnp.int32),
        pltpu.VMEM((_SLAB, _ROWS), jnp.float32),
        pltpu.VMEM((_SLAB, _ROWS), jnp.float32),
        pltpu.VMEM((16,), jnp.float32),
        pltpu.SemaphoreType.DMA,
        pltpu.SemaphoreType.DMA,
    ],
)


def kernel(lsm, target):
    lsmt = lsm.T
    tgt = target.astype(jnp.int32)
    sc_partials = _sc_call(lsmt, tgt)
    tc_partials = _tc_call(lsmt, tgt.reshape(1, _ROWS))
    return -(jnp.sum(sc_partials) + jnp.sum(tc_partials)) / _ROWS


# speedup vs baseline: 3.4254x; 1.0327x over previous
import jax
import jax.numpy as jnp
from jax import lax
from jax.experimental import pallas as pl
from jax.experimental.pallas import tpu as pltpu
from jax.experimental.pallas import tpu_sc as plsc

_N_CLASSES = 100000
_SMOOTHING = 0.1
_CONFIDENCE = 1.0 - _SMOOTHING
_EPS = _SMOOTHING / (_N_CLASSES - 1)

_ROWS = 1024

_C_SC = 47104
_TC_BR = 2048
_TC_BLK0 = _C_SC // _TC_BR
_TC_NBLK = (_N_CLASSES - _C_SC + _TC_BR - 1) // _TC_BR

_NC = 2
_NS = 16
_NW = _NC * _NS
_CPW = _C_SC // _NW
_SLAB = 32
_NSLAB = _CPW // _SLAB
_RG = _ROWS // 16


def _tc_kernel(lsmt_ref, tgt_ref, out_ref):
    j = pl.program_id(0)
    blk = lsmt_ref[...]
    cls = jax.lax.broadcasted_iota(jnp.int32, (_TC_BR, _ROWS), 0) + (
        _C_SC + j * _TC_BR
    )
    blk0 = jnp.where(cls < _N_CLASSES, blk, 0.0)
    s = jnp.sum(blk0)
    tgt = tgt_ref[...]
    g = jnp.sum(jnp.where(cls == tgt, blk, 0.0))
    out_ref[...] = jnp.reshape(_EPS * s + (_CONFIDENCE - _EPS) * g, (1, 1, 1))


def _tc_call(lsmt, tgt2d):
    return pl.pallas_call(
        _tc_kernel,
        grid=(_TC_NBLK,),
        in_specs=[
            pl.BlockSpec((_TC_BR, _ROWS), lambda j: (_TC_BLK0 + j, 0)),
            pl.BlockSpec((1, _ROWS), lambda j: (0, 0)),
        ],
        out_specs=pl.BlockSpec((1, 1, 1), lambda j: (j, 0, 0)),
        out_shape=jax.ShapeDtypeStruct((_TC_NBLK, 1, 1), jnp.float32),
        compiler_params=pltpu.CompilerParams(
            dimension_semantics=("arbitrary",),
        ),
    )(lsmt, tgt2d)


def _process_slab(buf, tvbuf, c0, carry):
    def rg_body(rg, carry):
        acc_s, acc_g = carry
        t_slice = tvbuf[pl.ds(rg * 16, 16)]
        for col in range(_SLAB):
            v = buf[col, pl.ds(rg * 16, 16)]
            acc_s = acc_s + v
            acc_g = acc_g + jnp.where(t_slice == c0 + col, v, 0.0)
        return acc_s, acc_g
    return lax.fori_loop(0, _RG, rg_body, carry)


def _sc_body(lsmt_hbm, tgt_hbm, out_hbm, tvbuf, buf0, buf1, outbuf, sem0, sem1):
    c = lax.axis_index("c")
    s = lax.axis_index("s")
    wid = s * _NC + c
    cbase = wid * _CPW
    pltpu.sync_copy(tgt_hbm, tvbuf)

    def slab_src(k):
        c0 = pl.multiple_of(cbase + k * _SLAB, 8)
        return lsmt_hbm.at[pl.ds(c0, _SLAB), :], c0

    src0, _ = slab_src(0)
    src1, _ = slab_src(1)
    pltpu.make_async_copy(src0, buf0, sem0).start()
    pltpu.make_async_copy(src1, buf1, sem1).start()

    def pair_step(p, carry):
        k = p * 2

        src_a, c0_a = slab_src(k)
        pltpu.make_async_copy(src_a, buf0, sem0).wait()
        carry = _process_slab(buf0, tvbuf, c0_a, carry)

        @pl.when(k + 2 < _NSLAB)
        def _():
            src_n, _ = slab_src(k + 2)
            pltpu.make_async_copy(src_n, buf0, sem0).start()

        src_b, c0_b = slab_src(k + 1)
        pltpu.make_async_copy(src_b, buf1, sem1).wait()
        carry = _process_slab(buf1, tvbuf, c0_b, carry)

        @pl.when(k + 3 < _NSLAB)
        def _():
            src_n, _ = slab_src(k + 3)
            pltpu.make_async_copy(src_n, buf1, sem1).start()

        return carry

    zero = jnp.zeros((16,), jnp.float32)
    acc_s, acc_g = lax.fori_loop(0, _NSLAB // 2, pair_step, (zero, zero))
    outbuf[...] = _EPS * acc_s + (_CONFIDENCE - _EPS) * acc_g
    pltpu.sync_copy(outbuf, out_hbm.at[pl.ds(wid * 16, 16)])


_sc_call = pl.kernel(
    _sc_body,
    out_type=jax.ShapeDtypeStruct((_NW * 16,), jnp.float32),
    mesh=plsc.VectorSubcoreMesh(
        core_axis_name="c", subcore_axis_name="s", num_cores=_NC, num_subcores=_NS
    ),
    scratch_types=[
        pltpu.VMEM((_ROWS,), jnp.int32),
        pltpu.VMEM((_SLAB, _ROWS), jnp.float32),
        pltpu.VMEM((_SLAB, _ROWS), jnp.float32),
        pltpu.VMEM((16,), jnp.float32),
        pltpu.SemaphoreType.DMA,
        pltpu.SemaphoreType.DMA,
    ],
)


def kernel(lsm, target):
    lsmt = lsm.T
    tgt = target.astype(jnp.int32)
    sc_partials = _sc_call(lsmt, tgt)
    tc_partials = _tc_call(lsmt, tgt.reshape(1, _ROWS))
    return -(jnp.sum(sc_partials) + jnp.sum(tc_partials)) / _ROWS
